# baseline (speedup 1.0000x reference)
import functools

import jax
import jax.numpy as jnp
from jax.experimental import pallas as pl
from jax.experimental.pallas import tpu as pltpu

TOKENS = 16384
D2 = 4096
HALF = D2 // 2
G = 8
BR = 1024


def _fused_body(gi_ref, x_ref, tab_ref, q_ref, s_ref):
    x = x_ref[...]
    left = x[:, :HALF]
    right = x[:, HALF:]
    sig = 1.0 / (1.0 + jnp.exp(-right))
    sw = (sig * right) * left

    row0 = pl.program_id(0) * BR
    rows = row0 + jax.lax.broadcasted_iota(jnp.int32, (BR, 1), 0)
    gid = jnp.zeros((BR, 1), jnp.int32)
    for i in range(G):
        gid += (rows >= gi_ref[i]).astype(jnp.int32)
    scale = jnp.broadcast_to(tab_ref[G, :][None, :], (BR, HALF))
    for i in range(G):
        scale = jnp.where(gid == i, tab_ref[i, :][None, :], scale)
    out = sw * scale

    y_max = jnp.max(jnp.abs(out), axis=1, keepdims=True)
    y_max = jnp.maximum(y_max, 1e-10)
    ds = 127.0 / y_max
    q_ref[...] = jnp.round(out * ds).astype(jnp.int8)
    s_ref[...] = ds


@jax.jit
def _run(x, table, group_index):
    grid = (TOKENS // BR,)
    q, s = pl.pallas_call(
        _fused_body,
        grid=grid,
        in_specs=[
            pl.BlockSpec(memory_space=pltpu.SMEM),
            pl.BlockSpec((BR, D2), lambda i: (i, 0)),
            pl.BlockSpec((G + 1, HALF), lambda i: (0, 0)),
        ],
        out_specs=[
            pl.BlockSpec((BR, HALF), lambda i: (i, 0)),
            pl.BlockSpec((BR, 1), lambda i: (i, 0)),
        ],
        out_shape=[
            jax.ShapeDtypeStruct((TOKENS, HALF), jnp.int8),
            jax.ShapeDtypeStruct((TOKENS, 1), jnp.float32),
        ],
        compiler_params=pltpu.CompilerParams(
            dimension_semantics=("arbitrary",),
        ),
    )(group_index, x, table)
    return q, jnp.squeeze(s, axis=-1)


def kernel(x, smooth_scales, group_index, quant_mode):
    table = jnp.concatenate(
        [smooth_scales.astype(jnp.float32),
         jnp.ones((1, HALF), jnp.float32)], axis=0)
    return _run(x, table, group_index)


# speedup vs baseline: 1.1764x; 1.1764x over previous
import functools

import jax
import jax.numpy as jnp
from jax.experimental import pallas as pl
from jax.experimental.pallas import tpu as pltpu

TOKENS = 16384
D2 = 4096
HALF = D2 // 2
G = 8
BR = 1024


def _fused_body(gi_ref, x_ref, tab_ref, q_ref, s_ref):
    x = x_ref[...]
    left = x[:, :HALF]
    right = x[:, HALF:]
    sig = 0.5 * jnp.tanh(0.5 * right) + 0.5
    sw = (sig * right) * left

    row0 = pl.program_id(0) * BR
    rows = row0 + jax.lax.broadcasted_iota(jnp.int32, (BR, 1), 0)
    gid = jnp.zeros((BR, 1), jnp.int32)
    for i in range(G):
        gid += (rows >= gi_ref[i]).astype(jnp.int32)
    onehot = (gid == jax.lax.broadcasted_iota(jnp.int32, (BR, G + 1), 1))
    scale = jnp.dot(onehot.astype(jnp.float32), tab_ref[...],
                    preferred_element_type=jnp.float32)
    out = sw * scale

    y_max = jnp.max(jnp.abs(out), axis=1, keepdims=True)
    y_max = jnp.maximum(y_max, 1e-10)
    ds = 127.0 / y_max
    q_ref[...] = jnp.round(out * ds).astype(jnp.int8)
    s_ref[...] = ds


@jax.jit
def _run(x, table, group_index):
    grid = (TOKENS // BR,)
    q, s = pl.pallas_call(
        _fused_body,
        grid=grid,
        in_specs=[
            pl.BlockSpec(memory_space=pltpu.SMEM),
            pl.BlockSpec((BR, D2), lambda i: (i, 0)),
            pl.BlockSpec((G + 1, HALF), lambda i: (0, 0)),
        ],
        out_specs=[
            pl.BlockSpec((BR, HALF), lambda i: (i, 0)),
            pl.BlockSpec((BR, 1), lambda i: (i, 0)),
        ],
        out_shape=[
            jax.ShapeDtypeStruct((TOKENS, HALF), jnp.int8),
            jax.ShapeDtypeStruct((TOKENS, 1), jnp.float32),
        ],
        compiler_params=pltpu.CompilerParams(
            dimension_semantics=("arbitrary",),
        ),
    )(group_index, x, table)
    return q, jnp.squeeze(s, axis=-1)


def kernel(x, smooth_scales, group_index, quant_mode):
    table = jnp.concatenate(
        [smooth_scales.astype(jnp.float32),
         jnp.ones((1, HALF), jnp.float32)], axis=0)
    return _run(x, table, group_index)
